# baseline (speedup 1.0000x reference)
import jax
import jax.numpy as jnp
from jax.experimental import pallas as pl
from jax.experimental.pallas import tpu as pltpu

_DN = (((1,), (0,)), ((), ()))


def _pick_bm(n, target):
    best = 8
    for bm in range(8, target + 1, 8):
        if n % bm == 0:
            best = bm
    return best


def _dot(a, b):
    return jax.lax.dot_general(
        a, b, _DN,
        precision=jax.lax.Precision.DEFAULT,
        preferred_element_type=jnp.float32,
    )


def _make_fused_kernel(bm):
    def _fused(x_ref, adj_ref, w1_ref, b1_ref, w2_ref, b2_ref, o_ref,
               s1_ref, s2_ref):
        p = pl.program_id(0)
        i = pl.program_id(1)

        @pl.when((p == 0) & (i == 0))
        def _():
            s1_ref[...] = _dot(x_ref[...], w1_ref[...])

        @pl.when(p == 0)
        def _():
            acc = _dot(adj_ref[...], s1_ref[...])
            hblk = jnp.maximum(acc + b1_ref[...], 0.0)
            s2_ref[pl.ds(i * bm, bm), :] = _dot(hblk, w2_ref[...])

        @pl.when(p == 1)
        def _():
            logits = _dot(adj_ref[...], s2_ref[...]) + b2_ref[...]
            m = jnp.max(logits, axis=1, keepdims=True)
            e = logits - m
            o_ref[...] = e - jnp.log(jnp.sum(jnp.exp(e), axis=1, keepdims=True))

    return _fused


def kernel(x, adj, W1, b1, W2, b2):
    n, nf = x.shape
    nh = W1.shape[1]
    nc = W2.shape[1]
    bm = _pick_bm(n, 400)
    nblk = n // bm

    def _adj_idx(p, i):
        return ((1 - p) * i + p * (nblk - 1 - i), 0)

    def _out_idx(p, i):
        return (p * (nblk - 1 - i), 0)

    return pl.pallas_call(
        _make_fused_kernel(bm),
        grid=(2, nblk),
        in_specs=[
            pl.BlockSpec((n, nf), lambda p, i: (0, 0)),
            pl.BlockSpec((bm, n), _adj_idx),
            pl.BlockSpec((nf, nh), lambda p, i: (0, 0)),
            pl.BlockSpec((1, nh), lambda p, i: (0, 0)),
            pl.BlockSpec((nh, nc), lambda p, i: (0, 0)),
            pl.BlockSpec((1, nc), lambda p, i: (0, 0)),
        ],
        out_specs=pl.BlockSpec((bm, nc), _out_idx),
        out_shape=jax.ShapeDtypeStruct((n, nc), jnp.float32),
        scratch_shapes=[
            pltpu.VMEM((n, nh), jnp.float32),
            pltpu.VMEM((n, nc), jnp.float32),
        ],
        compiler_params=pltpu.CompilerParams(
            dimension_semantics=("arbitrary", "arbitrary")
        ),
    )(x, adj, W1, b1.reshape(1, nh), W2, b2.reshape(1, nc))
